# Initial kernel scaffold; baseline (speedup 1.0000x reference)
import jax
import jax.numpy as jnp
from jax.experimental import pallas as pl


def kernel(x, edge_index, Wl1, bl1, Wr1, Wl2, bl2, Wr2, Wc, bc):
    raise NotImplementedError("write your pallas kernel here")


# speedup vs baseline: 10.7679x; 10.7679x over previous
import functools

import jax
import jax.numpy as jnp
from jax import lax
from jax.experimental import pallas as pl
from jax.experimental.pallas import tpu as pltpu
from jax.experimental.pallas import tpu_sc as plsc

N = 10000
E = 320000
D = 128
H1 = 128
H2 = 64
C = 40

NC = 2
NS = 16
NW = NC * NS
CH = 400
EPT = E // NW
CHUNKS = EPT // CH
W = 64
RPT = N // NS
ZR = 125
CNTP = 10240

BN = 512


def _front_body(x_ref, wl_ref, wr_ref, bl_ref, ya_ref, yb_ref, z_ref):
    xb = x_ref[...]
    y = jnp.dot(xb, wl_ref[...], preferred_element_type=jnp.float32)
    ya_ref[...] = y[:, :W]
    yb_ref[...] = y[:, W:]
    z_ref[...] = (jnp.dot(xb, wr_ref[...], preferred_element_type=jnp.float32)
                  + bl_ref[...])


def _front(x, Wl, Wr, bl):
    return pl.pallas_call(
        _front_body,
        grid=((N + BN - 1) // BN,),
        in_specs=[
            pl.BlockSpec((BN, D), lambda i: (i, 0)),
            pl.BlockSpec((D, H1), lambda i: (0, 0)),
            pl.BlockSpec((D, H1), lambda i: (0, 0)),
            pl.BlockSpec((H1,), lambda i: (0,)),
        ],
        out_specs=[pl.BlockSpec((BN, W), lambda i: (i, 0)),
                   pl.BlockSpec((BN, W), lambda i: (i, 0)),
                   pl.BlockSpec((BN, H1), lambda i: (i, 0))],
        out_shape=[jax.ShapeDtypeStruct((N, W), jnp.float32),
                   jax.ShapeDtypeStruct((N, W), jnp.float32),
                   jax.ShapeDtypeStruct((N, H1), jnp.float32)],
    )(x, Wl, Wr, bl)


def _mid_body(pa_ref, pb_ref, c_ref, z_ref, wl_ref, wr_ref, bl_ref,
              y2_ref, z2_ref):
    cnt = jnp.sum(c_ref[...], axis=0)
    inv = (1.0 / jnp.maximum(cnt, 1.0))[:, None]
    mean = jnp.concatenate([(pa_ref[0] + pa_ref[1]) * inv,
                            (pb_ref[0] + pb_ref[1]) * inv], axis=1)
    h = jnp.maximum(mean + z_ref[...], 0.0)
    y2_ref[...] = jnp.dot(h, wl_ref[...], preferred_element_type=jnp.float32)
    z2_ref[...] = (jnp.dot(h, wr_ref[...], preferred_element_type=jnp.float32)
                   + bl_ref[...])


def _mid(pa, pb, cnts, z1, Wl, Wr, bl):
    return pl.pallas_call(
        _mid_body,
        grid=((N + BN - 1) // BN,),
        in_specs=[
            pl.BlockSpec((2, BN, W), lambda i: (0, i, 0)),
            pl.BlockSpec((2, BN, W), lambda i: (0, i, 0)),
            pl.BlockSpec((NW, BN), lambda i: (0, i)),
            pl.BlockSpec((BN, H1), lambda i: (i, 0)),
            pl.BlockSpec((H1, H2), lambda i: (0, 0)),
            pl.BlockSpec((H1, H2), lambda i: (0, 0)),
            pl.BlockSpec((H2,), lambda i: (0,)),
        ],
        out_specs=[pl.BlockSpec((BN, H2), lambda i: (i, 0)),
                   pl.BlockSpec((BN, H2), lambda i: (i, 0))],
        out_shape=[jax.ShapeDtypeStruct((N, H2), jnp.float32),
                   jax.ShapeDtypeStruct((N, H2), jnp.float32)],
    )(pa, pb, cnts, z1, Wl, Wr, bl)


def _back_body(q_ref, c_ref, z_ref, wc_ref, bc_ref, o_ref):
    cnt = jnp.sum(c_ref[...], axis=0)
    inv = 1.0 / jnp.maximum(cnt, 1.0)
    emb = jnp.maximum((q_ref[0] + q_ref[1]) * inv[:, None] + z_ref[...], 0.0)
    o_ref[...] = (jnp.dot(emb, wc_ref[...], preferred_element_type=jnp.float32)
                  + bc_ref[...])


def _back(parts, cnts, z2, Wc, bc):
    return pl.pallas_call(
        _back_body,
        grid=((N + BN - 1) // BN,),
        in_specs=[
            pl.BlockSpec((2, BN, H2), lambda i: (0, i, 0)),
            pl.BlockSpec((NW, BN), lambda i: (0, i)),
            pl.BlockSpec((BN, H2), lambda i: (i, 0)),
            pl.BlockSpec((H2, C), lambda i: (0, 0)),
            pl.BlockSpec((C,), lambda i: (0,)),
        ],
        out_specs=pl.BlockSpec((BN, C), lambda i: (i, 0)),
        out_shape=jax.ShapeDtypeStruct((N, C), jnp.float32),
    )(parts, cnts, z2, Wc, bc)


def _sc_body(*refs):
    (y_hbm, src_hbm, dst_hbm, acc_out, cnt_out,
     src_v, dst_v, rows0, rows1, zbuf_v, cnt_v, acc_sh,
     sem0, sem1, ssem0, ssem1) = refs
    rows = (rows0, rows1)
    sems = (sem0, sem1)
    ssems = (ssem0, ssem1)

    cid = lax.axis_index("c")
    sid = lax.axis_index("s")
    zeros16 = jnp.zeros((16,), jnp.float32)
    ones16 = jnp.ones((16,), jnp.float32)

    def _zrow(i, _):
        def _zseg(j, _):
            zbuf_v[i, pl.ds(j * 16, 16)] = zeros16
            return 0
        lax.fori_loop(0, W // 16, _zseg, 0)
        return 0
    lax.fori_loop(0, ZR, _zrow, 0)
    for r in range(RPT // ZR):
        pltpu.sync_copy(zbuf_v, acc_sh.at[pl.ds(sid * RPT + r * ZR, ZR)])

    def _zc(i, _):
        cnt_v[pl.ds(i * 16, 16)] = zeros16
        return 0
    lax.fori_loop(0, CNTP // 16, _zc, 0)

    wid = cid * NS + sid
    pltpu.sync_copy(src_hbm.at[wid], src_v)
    pltpu.sync_copy(dst_hbm.at[wid], dst_v)

    plsc.subcore_barrier()

    def _cnt_chunk(k):
        def _cs(j, _):
            idx = dst_v[k, pl.ds(j * 16, 16)]
            plsc.addupdate_scatter(cnt_v, [idx], ones16)
            return 0
        lax.fori_loop(0, CH // 16, _cs, 0)

    NB = len(rows)

    def _quad(g, _):
        base = NB * g
        cps = [pltpu.async_copy(y_hbm.at[src_v.at[base + b]], rows[b], sems[b])
               for b in range(NB)]
        scs = []
        for b in range(NB):
            _cnt_chunk(base + b)
            cps[b].wait()
            scs.append(pltpu.async_copy(
                rows[b], acc_sh.at[dst_v.at[base + b]], ssems[b], add=True))
        for sc in scs:
            sc.wait()
        return 0
    lax.fori_loop(0, CHUNKS // NB, _quad, 0)
    for k in range(CHUNKS - CHUNKS % NB, CHUNKS):
        cp = pltpu.async_copy(y_hbm.at[src_v.at[k]], rows[0], sems[0])
        _cnt_chunk(k)
        cp.wait()
        pltpu.sync_copy(rows[0], acc_sh.at[dst_v.at[k]], add=True)

    plsc.subcore_barrier()

    pltpu.sync_copy(acc_sh.at[pl.ds(sid * RPT, RPT)],
                    acc_out.at[cid, pl.ds(sid * RPT, RPT)])
    pltpu.sync_copy(cnt_v, cnt_out.at[wid])


def _sc_segsum(y, src2d, dst2d):
    mesh = plsc.VectorSubcoreMesh(core_axis_name="c", subcore_axis_name="s")
    out_type = (jax.ShapeDtypeStruct((NC, N, W), jnp.float32),
                jax.ShapeDtypeStruct((NW, CNTP), jnp.float32))
    scratch = [
        pltpu.VMEM((CHUNKS, CH), jnp.int32),
        pltpu.VMEM((CHUNKS, CH), jnp.int32),
        pltpu.VMEM((CH, W), jnp.float32),
        pltpu.VMEM((CH, W), jnp.float32),
        pltpu.VMEM((ZR, W), jnp.float32),
        pltpu.VMEM((CNTP,), jnp.float32),
        pltpu.VMEM_SHARED((N, W), jnp.float32),
        pltpu.SemaphoreType.DMA,
        pltpu.SemaphoreType.DMA,
        pltpu.SemaphoreType.DMA,
        pltpu.SemaphoreType.DMA,
    ]

    kfn = functools.partial(
        pl.kernel,
        out_type=out_type,
        mesh=mesh,
        scratch_types=scratch,
        compiler_params=pltpu.CompilerParams(use_tc_tiling_on_sc=False,
                                             needs_layout_passes=False),
    )(_sc_body)
    return kfn(y, src2d, dst2d)


def kernel(x, edge_index, Wl1, bl1, Wr1, Wl2, bl2, Wr2, Wc, bc):
    src2d = edge_index[0].reshape(NW, CHUNKS, CH)
    dst2d = edge_index[1].reshape(NW, CHUNKS, CH)

    y1a, y1b, z1 = _front(x, Wl1, Wr1, bl1)
    agg1a, cnt = _sc_segsum(y1a, src2d, dst2d)
    agg1b, _ = _sc_segsum(y1b, src2d, dst2d)
    y2, z2 = _mid(agg1a, agg1b, cnt, z1, Wl2, Wr2, bl2)
    agg2, _ = _sc_segsum(y2, src2d, dst2d)
    return _back(agg2, cnt, z2, Wc, bc)


# speedup vs baseline: 10.8138x; 1.0043x over previous
import functools

import jax
import jax.numpy as jnp
from jax import lax
from jax.experimental import pallas as pl
from jax.experimental.pallas import tpu as pltpu
from jax.experimental.pallas import tpu_sc as plsc

N = 10000
E = 320000
D = 128
H1 = 128
H2 = 64
C = 40

NC = 2
NS = 16
NW = NC * NS
EPT = E // NW
W = 64
RPT = N // NS
ZR = 125
CNTP = 10240

CH1 = 80
CHUNKS1 = EPT // CH1
CH2 = 400
CHUNKS2 = EPT // CH2

BN = 512

_SC_PARAMS = pltpu.CompilerParams(use_tc_tiling_on_sc=False,
                                  needs_layout_passes=False)
_MESH = dict(core_axis_name="c", subcore_axis_name="s")


def _front_body(x_ref, wl_ref, wr_ref, bl_ref, ya_ref, yb_ref, z_ref):
    xb = x_ref[...]
    y = jnp.dot(xb, wl_ref[...], preferred_element_type=jnp.float32)
    ya_ref[...] = y[:, :W]
    yb_ref[...] = y[:, W:]
    z_ref[...] = (jnp.dot(xb, wr_ref[...], preferred_element_type=jnp.float32)
                  + bl_ref[...])


def _front(x, Wl, Wr, bl):
    return pl.pallas_call(
        _front_body,
        grid=((N + BN - 1) // BN,),
        in_specs=[
            pl.BlockSpec((BN, D), lambda i: (i, 0)),
            pl.BlockSpec((D, H1), lambda i: (0, 0)),
            pl.BlockSpec((D, H1), lambda i: (0, 0)),
            pl.BlockSpec((H1,), lambda i: (0,)),
        ],
        out_specs=[pl.BlockSpec((BN, W), lambda i: (i, 0)),
                   pl.BlockSpec((BN, W), lambda i: (i, 0)),
                   pl.BlockSpec((BN, H1), lambda i: (i, 0))],
        out_shape=[jax.ShapeDtypeStruct((N, W), jnp.float32),
                   jax.ShapeDtypeStruct((N, W), jnp.float32),
                   jax.ShapeDtypeStruct((N, H1), jnp.float32)],
    )(x, Wl, Wr, bl)


def _mid_body(pa_ref, pb_ref, c_ref, z_ref, wl_ref, wr_ref, bl_ref,
              y2_ref, z2_ref):
    cnt = jnp.sum(c_ref[...], axis=0)
    inv = (1.0 / jnp.maximum(cnt, 1.0))[:, None]
    mean = jnp.concatenate([(pa_ref[0] + pa_ref[1]) * inv,
                            (pb_ref[0] + pb_ref[1]) * inv], axis=1)
    h = jnp.maximum(mean + z_ref[...], 0.0)
    y2_ref[...] = jnp.dot(h, wl_ref[...], preferred_element_type=jnp.float32)
    z2_ref[...] = (jnp.dot(h, wr_ref[...], preferred_element_type=jnp.float32)
                   + bl_ref[...])


def _mid(pa, pb, cnts, z1, Wl, Wr, bl):
    return pl.pallas_call(
        _mid_body,
        grid=((N + BN - 1) // BN,),
        in_specs=[
            pl.BlockSpec((2, BN, W), lambda i: (0, i, 0)),
            pl.BlockSpec((2, BN, W), lambda i: (0, i, 0)),
            pl.BlockSpec((NW, BN), lambda i: (0, i)),
            pl.BlockSpec((BN, H1), lambda i: (i, 0)),
            pl.BlockSpec((H1, H2), lambda i: (0, 0)),
            pl.BlockSpec((H1, H2), lambda i: (0, 0)),
            pl.BlockSpec((H2,), lambda i: (0,)),
        ],
        out_specs=[pl.BlockSpec((BN, H2), lambda i: (i, 0)),
                   pl.BlockSpec((BN, H2), lambda i: (i, 0))],
        out_shape=[jax.ShapeDtypeStruct((N, H2), jnp.float32),
                   jax.ShapeDtypeStruct((N, H2), jnp.float32)],
    )(pa, pb, cnts, z1, Wl, Wr, bl)


def _back_body(q_ref, c_ref, z_ref, wc_ref, bc_ref, o_ref):
    cnt = jnp.sum(c_ref[...], axis=0)
    inv = 1.0 / jnp.maximum(cnt, 1.0)
    q = q_ref[0] + q_ref[1]
    emb = jnp.maximum(q * inv[:, None] + z_ref[...], 0.0)
    o_ref[...] = (jnp.dot(emb, wc_ref[...], preferred_element_type=jnp.float32)
                  + bc_ref[...])


def _back(parts, cnts, z2, Wc, bc):
    return pl.pallas_call(
        _back_body,
        grid=((N + BN - 1) // BN,),
        in_specs=[
            pl.BlockSpec((2, BN, H2), lambda i: (0, i, 0)),
            pl.BlockSpec((NW, BN), lambda i: (0, i)),
            pl.BlockSpec((BN, H2), lambda i: (i, 0)),
            pl.BlockSpec((H2, C), lambda i: (0, 0)),
            pl.BlockSpec((C,), lambda i: (0,)),
        ],
        out_specs=pl.BlockSpec((BN, C), lambda i: (i, 0)),
        out_shape=jax.ShapeDtypeStruct((N, C), jnp.float32),
    )(parts, cnts, z2, Wc, bc)


def _zero_fill(zbuf_v, acc_sh, sid):
    for r in range(RPT // ZR):
        pltpu.sync_copy(zbuf_v, acc_sh.at[pl.ds(sid * RPT + r * ZR, ZR)])


def _zero_vmem_2d(buf, nrow, ncol):
    zeros16 = jnp.zeros((16,), jnp.float32)

    def _zrow(i, _):
        def _zseg(j, _):
            buf[i, pl.ds(j * 16, 16)] = zeros16
            return 0
        lax.fori_loop(0, ncol // 16, _zseg, 0)
        return 0
    lax.fori_loop(0, nrow, _zrow, 0)


def _make_cnt_chunk(dst_v, cnt_v, ch):
    ones16 = jnp.ones((16,), jnp.float32)

    def _cnt_chunk(k):
        def _cs(j, _):
            idx = dst_v[k, pl.ds(j * 16, 16)]
            plsc.addupdate_scatter(cnt_v, [idx], ones16)
            return 0
        lax.fori_loop(0, ch // 16, _cs, 0)
    return _cnt_chunk


def _sc1_body(ya_hbm, yb_hbm, src_hbm, dst_hbm, acca_out, accb_out, cnt_out,
              src_v, dst_v, rows0, rows1, zbuf_v, cnt_v, acc_sh,
              g0, g1, s0, s1):
    rows = (rows0, rows1)
    gsems = (g0, g1)
    ssems = (s0, s1)

    cid = lax.axis_index("c")
    sid = lax.axis_index("s")

    _zero_vmem_2d(zbuf_v, ZR, W)
    _zero_fill(zbuf_v, acc_sh, sid)

    def _zc(i, _):
        cnt_v[pl.ds(i * 16, 16)] = jnp.zeros((16,), jnp.float32)
        return 0
    lax.fori_loop(0, CNTP // 16, _zc, 0)

    wid = cid * NS + sid
    pltpu.sync_copy(src_hbm.at[wid], src_v)
    pltpu.sync_copy(dst_hbm.at[wid], dst_v)

    plsc.subcore_barrier()

    cnt_chunk = _make_cnt_chunk(dst_v, cnt_v, CH2)
    NB = len(rows)

    def _run_phase(y_hbm, acc_sh, with_cnt):
        def _pair(g, _):
            base = NB * g
            cps = [pltpu.async_copy(y_hbm.at[src_v.at[base + b]], rows[b],
                                    gsems[b]) for b in range(NB)]
            scs = []
            for b in range(NB):
                if with_cnt:
                    cnt_chunk(base + b)
                cps[b].wait()
                scs.append(pltpu.async_copy(
                    rows[b], acc_sh.at[dst_v.at[base + b]], ssems[b],
                    add=True))
            for sc in scs:
                sc.wait()
            return 0
        lax.fori_loop(0, CHUNKS2 // NB, _pair, 0)
        for k in range(CHUNKS2 - CHUNKS2 % NB, CHUNKS2):
            cp = pltpu.async_copy(y_hbm.at[src_v.at[k]], rows[0], gsems[0])
            if with_cnt:
                cnt_chunk(k)
            cp.wait()
            pltpu.sync_copy(rows[0], acc_sh.at[dst_v.at[k]], add=True)

    _run_phase(ya_hbm, acc_sh, True)

    plsc.subcore_barrier()
    pltpu.sync_copy(acc_sh.at[pl.ds(sid * RPT, RPT)],
                    acca_out.at[cid, pl.ds(sid * RPT, RPT)])
    _zero_fill(zbuf_v, acc_sh, sid)
    pltpu.sync_copy(cnt_v, cnt_out.at[wid])
    plsc.subcore_barrier()

    _run_phase(yb_hbm, acc_sh, False)

    plsc.subcore_barrier()
    pltpu.sync_copy(acc_sh.at[pl.ds(sid * RPT, RPT)],
                    accb_out.at[cid, pl.ds(sid * RPT, RPT)])


def _sc_pass1(ya, yb, src2d, dst2d):
    mesh = plsc.VectorSubcoreMesh(**_MESH)
    out_type = (jax.ShapeDtypeStruct((NC, N, W), jnp.float32),
                jax.ShapeDtypeStruct((NC, N, W), jnp.float32),
                jax.ShapeDtypeStruct((NW, CNTP), jnp.float32))
    scratch = [
        pltpu.VMEM((CHUNKS2, CH2), jnp.int32),
        pltpu.VMEM((CHUNKS2, CH2), jnp.int32),
        pltpu.VMEM((CH2, W), jnp.float32),
        pltpu.VMEM((CH2, W), jnp.float32),
        pltpu.VMEM((ZR, W), jnp.float32),
        pltpu.VMEM((CNTP,), jnp.float32),
        pltpu.VMEM_SHARED((N, W), jnp.float32),
    ] + [pltpu.SemaphoreType.DMA] * 4
    kfn = functools.partial(
        pl.kernel, out_type=out_type, mesh=mesh, scratch_types=scratch,
        compiler_params=_SC_PARAMS,
    )(_sc1_body)
    return kfn(ya, yb, src2d, dst2d)


def _sc2_body(y_hbm, src_hbm, dst_hbm, acc_out, cnt_out,
              src_v, dst_v, rows0, rows1, zbuf_v, cnt_v, acc_sh,
              g0, g1, s0, s1):
    rows = (rows0, rows1)
    gsems = (g0, g1)
    ssems = (s0, s1)

    cid = lax.axis_index("c")
    sid = lax.axis_index("s")

    _zero_vmem_2d(zbuf_v, ZR, W)
    _zero_fill(zbuf_v, acc_sh, sid)

    def _zc(i, _):
        cnt_v[pl.ds(i * 16, 16)] = jnp.zeros((16,), jnp.float32)
        return 0
    lax.fori_loop(0, CNTP // 16, _zc, 0)

    wid = cid * NS + sid
    pltpu.sync_copy(src_hbm.at[wid], src_v)
    pltpu.sync_copy(dst_hbm.at[wid], dst_v)

    plsc.subcore_barrier()

    cnt_chunk = _make_cnt_chunk(dst_v, cnt_v, CH2)
    NB = len(rows)

    def _pair(g, _):
        base = NB * g
        cps = [pltpu.async_copy(y_hbm.at[src_v.at[base + b]], rows[b],
                                gsems[b]) for b in range(NB)]
        scs = []
        for b in range(NB):
            cnt_chunk(base + b)
            cps[b].wait()
            scs.append(pltpu.async_copy(
                rows[b], acc_sh.at[dst_v.at[base + b]], ssems[b], add=True))
        for sc in scs:
            sc.wait()
        return 0
    lax.fori_loop(0, CHUNKS2 // NB, _pair, 0)
    for k in range(CHUNKS2 - CHUNKS2 % NB, CHUNKS2):
        cp = pltpu.async_copy(y_hbm.at[src_v.at[k]], rows[0], gsems[0])
        cnt_chunk(k)
        cp.wait()
        pltpu.sync_copy(rows[0], acc_sh.at[dst_v.at[k]], add=True)

    plsc.subcore_barrier()

    pltpu.sync_copy(acc_sh.at[pl.ds(sid * RPT, RPT)],
                    acc_out.at[cid, pl.ds(sid * RPT, RPT)])
    pltpu.sync_copy(cnt_v, cnt_out.at[wid])


def _sc_pass2(y2, src2d, dst2d):
    mesh = plsc.VectorSubcoreMesh(**_MESH)
    out_type = (jax.ShapeDtypeStruct((NC, N, W), jnp.float32),
                jax.ShapeDtypeStruct((NW, CNTP), jnp.float32))
    scratch = [
        pltpu.VMEM((CHUNKS2, CH2), jnp.int32),
        pltpu.VMEM((CHUNKS2, CH2), jnp.int32),
        pltpu.VMEM((CH2, W), jnp.float32),
        pltpu.VMEM((CH2, W), jnp.float32),
        pltpu.VMEM((ZR, W), jnp.float32),
        pltpu.VMEM((CNTP,), jnp.float32),
        pltpu.VMEM_SHARED((N, W), jnp.float32),
    ] + [pltpu.SemaphoreType.DMA] * 4
    kfn = functools.partial(
        pl.kernel, out_type=out_type, mesh=mesh, scratch_types=scratch,
        compiler_params=_SC_PARAMS,
    )(_sc2_body)
    return kfn(y2, src2d, dst2d)


def kernel(x, edge_index, Wl1, bl1, Wr1, Wl2, bl2, Wr2, Wc, bc):
    src2 = edge_index[0].reshape(NW, CHUNKS2, CH2)
    dst2 = edge_index[1].reshape(NW, CHUNKS2, CH2)

    y1a, y1b, z1 = _front(x, Wl1, Wr1, bl1)
    agg1a, agg1b, cnt = _sc_pass1(y1a, y1b, src2, dst2)
    y2, z2 = _mid(agg1a, agg1b, cnt, z1, Wl2, Wr2, bl2)
    agg2, _ = _sc_pass2(y2, src2, dst2)
    return _back(agg2, cnt, z2, Wc, bc)


# speedup vs baseline: 12.0539x; 1.1147x over previous
import functools

import jax
import jax.numpy as jnp
from jax import lax
from jax.experimental import pallas as pl
from jax.experimental.pallas import tpu as pltpu
from jax.experimental.pallas import tpu_sc as plsc

N = 10000
E = 320000
D = 128
H1 = 128
H2 = 64
C = 40

NC = 2
NS = 16
NW = NC * NS
EPT = E // NW
W = 64
RPT = N // NS
ZR = 125
CNTP = 10240

CH2 = 400
CHUNKS2 = EPT // CH2

BN = 512

_SC_PARAMS = pltpu.CompilerParams(use_tc_tiling_on_sc=False,
                                  needs_layout_passes=False)
_MESH = dict(core_axis_name="c", subcore_axis_name="s")


def _front_body(x_ref, wl_ref, wr_ref, bl_ref, ya_ref, yb_ref, z_ref):
    xb = x_ref[...]
    y = jnp.dot(xb, wl_ref[...], preferred_element_type=jnp.float32)
    ya_ref[...] = y[:, :W]
    yb_ref[...] = y[:, W:]
    z_ref[...] = (jnp.dot(xb, wr_ref[...], preferred_element_type=jnp.float32)
                  + bl_ref[...])


def _front(x, Wl, Wr, bl):
    return pl.pallas_call(
        _front_body,
        grid=((N + BN - 1) // BN,),
        in_specs=[
            pl.BlockSpec((BN, D), lambda i: (i, 0)),
            pl.BlockSpec((D, H1), lambda i: (0, 0)),
            pl.BlockSpec((D, H1), lambda i: (0, 0)),
            pl.BlockSpec((H1,), lambda i: (0,)),
        ],
        out_specs=[pl.BlockSpec((BN, W), lambda i: (i, 0)),
                   pl.BlockSpec((BN, W), lambda i: (i, 0)),
                   pl.BlockSpec((BN, H1), lambda i: (i, 0))],
        out_shape=[jax.ShapeDtypeStruct((N, W), jnp.float32),
                   jax.ShapeDtypeStruct((N, W), jnp.float32),
                   jax.ShapeDtypeStruct((N, H1), jnp.float32)],
    )(x, Wl, Wr, bl)


def _mid_body(p_ref, c_ref, z_ref, wl_ref, wr_ref, bl_ref,
              y2_ref, z2_ref):
    cnt = jnp.sum(c_ref[...], axis=0)
    inv = (1.0 / jnp.maximum(cnt, 1.0))[:, None]
    h = jnp.maximum((p_ref[0] + p_ref[1]) * inv + z_ref[...], 0.0)
    y2_ref[...] = jnp.dot(h, wl_ref[...], preferred_element_type=jnp.float32)
    z2_ref[...] = (jnp.dot(h, wr_ref[...], preferred_element_type=jnp.float32)
                   + bl_ref[...])


def _mid(p, cnts, z1, Wl, Wr, bl):
    return pl.pallas_call(
        _mid_body,
        grid=((N + BN - 1) // BN,),
        in_specs=[
            pl.BlockSpec((2, BN, H1), lambda i: (0, i, 0)),
            pl.BlockSpec((NW, BN), lambda i: (0, i)),
            pl.BlockSpec((BN, H1), lambda i: (i, 0)),
            pl.BlockSpec((H1, H2), lambda i: (0, 0)),
            pl.BlockSpec((H1, H2), lambda i: (0, 0)),
            pl.BlockSpec((H2,), lambda i: (0,)),
        ],
        out_specs=[pl.BlockSpec((BN, H2), lambda i: (i, 0)),
                   pl.BlockSpec((BN, H2), lambda i: (i, 0))],
        out_shape=[jax.ShapeDtypeStruct((N, H2), jnp.float32),
                   jax.ShapeDtypeStruct((N, H2), jnp.float32)],
    )(p, cnts, z1, Wl, Wr, bl)


def _back_body(q_ref, c_ref, z_ref, wc_ref, bc_ref, o_ref):
    cnt = jnp.sum(c_ref[...], axis=0)
    inv = 1.0 / jnp.maximum(cnt, 1.0)
    q = q_ref[0, :, :H2] + q_ref[1, :, :H2]
    emb = jnp.maximum(q * inv[:, None] + z_ref[...], 0.0)
    o_ref[...] = (jnp.dot(emb, wc_ref[...], preferred_element_type=jnp.float32)
                  + bc_ref[...])


def _back(parts, cnts, z2, Wc, bc):
    return pl.pallas_call(
        _back_body,
        grid=((N + BN - 1) // BN,),
        in_specs=[
            pl.BlockSpec((2, BN, H1), lambda i: (0, i, 0)),
            pl.BlockSpec((NW, BN), lambda i: (0, i)),
            pl.BlockSpec((BN, H2), lambda i: (i, 0)),
            pl.BlockSpec((H2, C), lambda i: (0, 0)),
            pl.BlockSpec((C,), lambda i: (0,)),
        ],
        out_specs=pl.BlockSpec((BN, C), lambda i: (i, 0)),
        out_shape=jax.ShapeDtypeStruct((N, C), jnp.float32),
    )(parts, cnts, z2, Wc, bc)


def _zero_fill(zbuf_v, acc_sh, sid):
    for r in range(RPT // ZR):
        pltpu.sync_copy(zbuf_v, acc_sh.at[pl.ds(sid * RPT + r * ZR, ZR)])


def _zero_vmem_2d(buf, nrow, ncol):
    zeros16 = jnp.zeros((16,), jnp.float32)

    def _zrow(i, _):
        def _zseg(j, _):
            buf[i, pl.ds(j * 16, 16)] = zeros16
            return 0
        lax.fori_loop(0, ncol // 16, _zseg, 0)
        return 0
    lax.fori_loop(0, nrow, _zrow, 0)


def _make_cnt_chunk(dst_v, cnt_v, ch):
    ones16 = jnp.ones((16,), jnp.float32)

    def _cnt_chunk(k):
        def _cs(j, _):
            idx = dst_v[pl.ds(k * ch + j * 16, 16)]
            plsc.addupdate_scatter(cnt_v, [idx], ones16)
            return 0
        lax.fori_loop(0, ch // 16, _cs, 0)
    return _cnt_chunk


def _sc1_body(ya_hbm, yb_hbm, edge_hbm, agg_out, cnt_out,
              src_v, dst_v, rows0, rows1, zbuf_v, cnt_v, acc_sh,
              g0, g1, s0, s1):
    rows = (rows0, rows1)
    gsems = (g0, g1)
    ssems = (s0, s1)

    cid = lax.axis_index("c")
    sid = lax.axis_index("s")

    _zero_vmem_2d(zbuf_v, ZR, W)
    _zero_fill(zbuf_v, acc_sh, sid)

    def _zc(i, _):
        cnt_v[pl.ds(i * 16, 16)] = jnp.zeros((16,), jnp.float32)
        return 0
    lax.fori_loop(0, CNTP // 16, _zc, 0)

    wid = cid * NS + sid
    pltpu.sync_copy(edge_hbm.at[0, pl.ds(wid * EPT, EPT)], src_v)
    pltpu.sync_copy(edge_hbm.at[1, pl.ds(wid * EPT, EPT)], dst_v)

    plsc.subcore_barrier()

    cnt_chunk = _make_cnt_chunk(dst_v, cnt_v, CH2)
    NB = len(rows)

    def _run_phase(y_hbm, acc_sh, with_cnt):
        def _pair(g, _):
            base = NB * g
            cps = [pltpu.async_copy(
                y_hbm.at[src_v.at[pl.ds((base + b) * CH2, CH2)]], rows[b],
                gsems[b]) for b in range(NB)]
            scs = []
            for b in range(NB):
                if with_cnt:
                    cnt_chunk(base + b)
                cps[b].wait()
                scs.append(pltpu.async_copy(
                    rows[b],
                    acc_sh.at[dst_v.at[pl.ds((base + b) * CH2, CH2)]],
                    ssems[b], add=True))
            for sc in scs:
                sc.wait()
            return 0
        lax.fori_loop(0, CHUNKS2 // NB, _pair, 0)
        for k in range(CHUNKS2 - CHUNKS2 % NB, CHUNKS2):
            cp = pltpu.async_copy(
                y_hbm.at[src_v.at[pl.ds(k * CH2, CH2)]], rows[0], gsems[0])
            if with_cnt:
                cnt_chunk(k)
            cp.wait()
            pltpu.sync_copy(rows[0], acc_sh.at[dst_v.at[pl.ds(k * CH2, CH2)]],
                            add=True)

    _run_phase(ya_hbm, acc_sh, True)

    plsc.subcore_barrier()
    pltpu.sync_copy(acc_sh.at[pl.ds(sid * RPT, RPT)],
                    agg_out.at[cid, pl.ds(sid * RPT, RPT), pl.ds(0, W)])
    _zero_fill(zbuf_v, acc_sh, sid)
    pltpu.sync_copy(cnt_v, cnt_out.at[wid])
    plsc.subcore_barrier()

    _run_phase(yb_hbm, acc_sh, False)

    plsc.subcore_barrier()
    pltpu.sync_copy(acc_sh.at[pl.ds(sid * RPT, RPT)],
                    agg_out.at[cid, pl.ds(sid * RPT, RPT), pl.ds(W, W)])


def _sc_pass1(ya, yb, edge_index):
    mesh = plsc.VectorSubcoreMesh(**_MESH)
    out_type = (jax.ShapeDtypeStruct((NC, N, H1), jnp.float32),
                jax.ShapeDtypeStruct((NW, CNTP), jnp.float32))
    scratch = [
        pltpu.VMEM((EPT,), jnp.int32),
        pltpu.VMEM((EPT,), jnp.int32),
        pltpu.VMEM((CH2, W), jnp.float32),
        pltpu.VMEM((CH2, W), jnp.float32),
        pltpu.VMEM((ZR, W), jnp.float32),
        pltpu.VMEM((CNTP,), jnp.float32),
        pltpu.VMEM_SHARED((N, W), jnp.float32),
    ] + [pltpu.SemaphoreType.DMA] * 4
    kfn = functools.partial(
        pl.kernel, out_type=out_type, mesh=mesh, scratch_types=scratch,
        compiler_params=_SC_PARAMS,
    )(_sc1_body)
    return kfn(ya, yb, edge_index)


def _sc2_body(y_hbm, edge_hbm, acc_out, cnt_out,
              src_v, dst_v, rows0, rows1, zbuf_v, cnt_v, acc_sh,
              g0, g1, s0, s1):
    rows = (rows0, rows1)
    gsems = (g0, g1)
    ssems = (s0, s1)

    cid = lax.axis_index("c")
    sid = lax.axis_index("s")

    _zero_vmem_2d(zbuf_v, ZR, W)
    _zero_fill(zbuf_v, acc_sh, sid)

    def _zc(i, _):
        cnt_v[pl.ds(i * 16, 16)] = jnp.zeros((16,), jnp.float32)
        return 0
    lax.fori_loop(0, CNTP // 16, _zc, 0)

    wid = cid * NS + sid
    pltpu.sync_copy(edge_hbm.at[0, pl.ds(wid * EPT, EPT)], src_v)
    pltpu.sync_copy(edge_hbm.at[1, pl.ds(wid * EPT, EPT)], dst_v)

    plsc.subcore_barrier()

    cnt_chunk = _make_cnt_chunk(dst_v, cnt_v, CH2)
    NB = len(rows)

    def _pair(g, _):
        base = NB * g
        cps = [pltpu.async_copy(
            y_hbm.at[src_v.at[pl.ds((base + b) * CH2, CH2)]], rows[b],
            gsems[b]) for b in range(NB)]
        scs = []
        for b in range(NB):
            cnt_chunk(base + b)
            cps[b].wait()
            scs.append(pltpu.async_copy(
                rows[b], acc_sh.at[dst_v.at[pl.ds((base + b) * CH2, CH2)]],
                ssems[b], add=True))
        for sc in scs:
            sc.wait()
        return 0
    lax.fori_loop(0, CHUNKS2 // NB, _pair, 0)
    for k in range(CHUNKS2 - CHUNKS2 % NB, CHUNKS2):
        cp = pltpu.async_copy(
            y_hbm.at[src_v.at[pl.ds(k * CH2, CH2)]], rows[0], gsems[0])
        cnt_chunk(k)
        cp.wait()
        pltpu.sync_copy(rows[0], acc_sh.at[dst_v.at[pl.ds(k * CH2, CH2)]],
                        add=True)

    plsc.subcore_barrier()

    pltpu.sync_copy(acc_sh.at[pl.ds(sid * RPT, RPT)],
                    acc_out.at[cid, pl.ds(sid * RPT, RPT), pl.ds(0, W)])
    pltpu.sync_copy(cnt_v, cnt_out.at[wid])


def _sc_pass2(y2, edge_index):
    mesh = plsc.VectorSubcoreMesh(**_MESH)
    out_type = (jax.ShapeDtypeStruct((NC, N, H1), jnp.float32),
                jax.ShapeDtypeStruct((NW, CNTP), jnp.float32))
    scratch = [
        pltpu.VMEM((EPT,), jnp.int32),
        pltpu.VMEM((EPT,), jnp.int32),
        pltpu.VMEM((CH2, W), jnp.float32),
        pltpu.VMEM((CH2, W), jnp.float32),
        pltpu.VMEM((ZR, W), jnp.float32),
        pltpu.VMEM((CNTP,), jnp.float32),
        pltpu.VMEM_SHARED((N, W), jnp.float32),
    ] + [pltpu.SemaphoreType.DMA] * 4
    kfn = functools.partial(
        pl.kernel, out_type=out_type, mesh=mesh, scratch_types=scratch,
        compiler_params=_SC_PARAMS,
    )(_sc2_body)
    return kfn(y2, edge_index)


def kernel(x, edge_index, Wl1, bl1, Wr1, Wl2, bl2, Wr2, Wc, bc):
    y1a, y1b, z1 = _front(x, Wl1, Wr1, bl1)
    agg1, cnt = _sc_pass1(y1a, y1b, edge_index)
    y2, z2 = _mid(agg1, cnt, z1, Wl2, Wr2, bl2)
    agg2, _ = _sc_pass2(y2, edge_index)
    return _back(agg2, cnt, z2, Wc, bc)
